# baseline (speedup 1.0000x reference)
import functools

import numpy as np
import jax
import jax.numpy as jnp
from jax import lax
from jax.experimental import pallas as pl
from jax.experimental.pallas import tpu as pltpu
from jax.experimental.pallas import tpu_sc as plsc

_N_LEVELS = 17
_F = 4
_LOG2_T = 21
_T = 1 << _LOG2_T
_BASE_RES = 8
_MAX_RES = 8192
_SCALE = np.exp((np.log(_MAX_RES) - np.log(_BASE_RES)) / (_N_LEVELS - 1))
_RES = [int(np.floor(_BASE_RES * _SCALE ** l)) for l in range(_N_LEVELS)]
_SIZES = [min(_T, (r + 1) ** 3) for r in _RES]
_OFFS = np.concatenate([[0], np.cumsum(_SIZES)]).astype(np.int64)
_PRIME1 = np.int32(-1640531535)
_PRIME2 = np.int32(805459861)
_N_DENSE = sum(1 for l in range(_N_LEVELS) if _SIZES[l] == (_RES[l] + 1) ** 3)

_NC, _NS, _L = 2, 16, 16
_NW = _NC * _NS
_P = 512
_CIDX = 8 * _P
_GCH = 128


def _sc_body(px_hbm, py_hbm, pz_hbm, table_hbm, resf_hbm, offi_hbm, enc_hbm,
             pxv, pyv, pzv, resv, offv, idxv, subv, wv, rowsv, encv, sem):
    wid = lax.axis_index("s") * _NC + lax.axis_index("c")
    npoints = px_hbm.shape[0]
    nper = npoints // _NW
    nchunks = nper // _P
    base = wid * nper

    pltpu.sync_copy(resf_hbm, resv)
    pltpu.sync_copy(offi_hbm, offv)

    iota = lax.iota(jnp.int32, _L)

    def floorv(v):
        vi = v.astype(jnp.int32)
        vf = vi.astype(jnp.float32)
        vi = vi - jnp.where(v < vf, 1, 0)
        return vi, v - vi.astype(jnp.float32)

    def chunk_body(ci, carry):
        cbase = base + ci * _P
        pltpu.sync_copy(px_hbm.at[pl.ds(cbase, _P)], pxv)
        pltpu.sync_copy(py_hbm.at[pl.ds(cbase, _P)], pyv)
        pltpu.sync_copy(pz_hbm.at[pl.ds(cbase, _P)], pzv)

        def do_level(l, r_f, off_i, dense):
            r_i = r_f.astype(jnp.int32)
            rp1 = r_i + 1

            def grp_idx(g, c):
                s = g * _L
                x = pxv[pl.ds(s, _L)] * r_f
                y = pyv[pl.ds(s, _L)] * r_f
                z = pzv[pl.ds(s, _L)] * r_f
                xi, fx = floorv(x)
                yi, fy = floorv(y)
                zi, fz = floorv(z)
                ox, oy, oz = 1.0 - fx, 1.0 - fy, 1.0 - fz
                for corner in range(8):
                    bx, by, bz = corner & 1, (corner >> 1) & 1, (corner >> 2) & 1
                    cx = xi + bx
                    cy = yi + by
                    cz = zi + bz
                    if dense:
                        cx2 = jnp.minimum(jnp.maximum(cx, 0), r_i)
                        cy2 = jnp.minimum(jnp.maximum(cy, 0), r_i)
                        cz2 = jnp.minimum(jnp.maximum(cz, 0), r_i)
                        idx = cx2 + rp1 * (cy2 + rp1 * cz2)
                    else:
                        idx = (cx ^ (cy * _PRIME1) ^ (cz * _PRIME2)) & jnp.int32(_T - 1)
                    idx = idx + off_i
                    w = (fx if bx else ox) * (fy if by else oy) * (fz if bz else oz)
                    flat = corner * _P + s
                    idxv[flat // _GCH, pl.ds(flat % _GCH, _L)] = idx >> 1
                    subv[pl.ds(flat, _L)] = (idx & 1) * 4
                    wv[pl.ds(flat, _L)] = w
                return c

            lax.fori_loop(0, _P // _L, grp_idx, 0)

            descs = []
            for j in range(_CIDX // _GCH):
                descs.append(pltpu.async_copy(
                    table_hbm.at[idxv.at[j]],
                    rowsv.at[pl.ds(j * _GCH, _GCH)], sem))
            for d in descs:
                d.wait()

            def grp_acc(g, c):
                s = g * _L
                acc = [jnp.zeros((_L,), jnp.float32) for _ in range(_F)]
                for corner in range(8):
                    flat = corner * _P + s
                    wc = wv[pl.ds(flat, _L)]
                    sub = subv[pl.ds(flat, _L)]
                    rids = iota + flat
                    for f in range(_F):
                        feat = plsc.load_gather(rowsv, [rids, sub + f])
                        acc[f] = acc[f] + wc * feat
                pids = iota + s
                for f in range(_F):
                    plsc.store_scatter(encv, [pids, jnp.broadcast_to(l * _F + f, (_L,))],
                                       acc[f])
                return c

            lax.fori_loop(0, _P // _L, grp_acc, 0)

        def dense_level(l, c):
            do_level(l, resv[pl.ds(l * _L, _L)], offv[pl.ds(l * _L, _L)], True)
            return c

        def hash_level(l, c):
            do_level(l, resv[pl.ds(l * _L, _L)], offv[pl.ds(l * _L, _L)], False)
            return c

        lax.fori_loop(0, _N_DENSE, dense_level, 0)
        lax.fori_loop(_N_DENSE, _N_LEVELS, hash_level, 0)

        pltpu.sync_copy(encv, enc_hbm.at[pl.ds(cbase, _P)])
        return carry

    lax.fori_loop(0, nchunks, chunk_body, 0)


def _encode_sc(px, py, pz, table2):
    n = px.shape[0]
    enc_dim = _N_LEVELS * _F
    resf = jnp.asarray(np.repeat(np.array(_RES, np.float32), _L))
    offi = jnp.asarray(np.repeat(_OFFS[:_N_LEVELS].astype(np.int32), _L))
    mesh = plsc.VectorSubcoreMesh(core_axis_name="c", subcore_axis_name="s")
    f = pl.kernel(
        _sc_body,
        out_type=jax.ShapeDtypeStruct((n, enc_dim), jnp.float32),
        mesh=mesh,
        compiler_params=pltpu.CompilerParams(
            needs_layout_passes=False, use_tc_tiling_on_sc=False),
        scratch_types=[
            pltpu.VMEM((_P,), jnp.float32),
            pltpu.VMEM((_P,), jnp.float32),
            pltpu.VMEM((_P,), jnp.float32),
            pltpu.VMEM((_N_LEVELS * _L,), jnp.float32),
            pltpu.VMEM((_N_LEVELS * _L,), jnp.int32),
            pltpu.VMEM((_CIDX // _GCH, _GCH), jnp.int32),
            pltpu.VMEM((_CIDX,), jnp.int32),
            pltpu.VMEM((_CIDX,), jnp.float32),
            pltpu.VMEM((_CIDX, 8), jnp.float32),
            pltpu.VMEM((_P, enc_dim), jnp.float32),
            pltpu.SemaphoreType.DMA,
        ],
    )
    return f(px, py, pz, table2, resf, offi)


def _mlp_body(enc_ref, w1_ref, w2_ref, o_ref):
    h = jnp.dot(enc_ref[...], w1_ref[...], preferred_element_type=jnp.float32)
    h = jnp.maximum(h, 0.0)
    o = jnp.dot(h, w2_ref[...], preferred_element_type=jnp.float32)
    col = lax.broadcasted_iota(jnp.int32, o.shape, 1)
    sig = 1.0 / (1.0 + jnp.exp(-o))
    o_ref[...] = jnp.where(col < 3, sig, o)


def _mlp(enc, w1, w2p):
    n, enc_dim = enc.shape
    blk = 4096
    return pl.pallas_call(
        _mlp_body,
        grid=(n // blk,),
        in_specs=[
            pl.BlockSpec((blk, enc_dim), lambda i: (i, 0)),
            pl.BlockSpec((enc_dim, 64), lambda i: (0, 0)),
            pl.BlockSpec((64, _F), lambda i: (0, 0)),
        ],
        out_specs=pl.BlockSpec((blk, _F), lambda i: (i, 0)),
        out_shape=jax.ShapeDtypeStruct((n, _F), jnp.float32),
    )(enc, w1, w2p)


def kernel(viewdirs, table, W1, W2):
    sb, b, _ = viewdirs.shape
    n = sb * b
    pos = viewdirs.reshape(n, 3)
    px, py, pz = pos[:, 0], pos[:, 1], pos[:, 2]
    table2 = table.reshape(table.shape[0] // 2, 2 * _F)
    enc = _encode_sc(px, py, pz, table2)
    w2p = jnp.concatenate([W2[:, 1:], W2[:, :1]], axis=1)
    out = _mlp(enc, W1, w2p)
    return out.reshape(sb, b, _F)
